# baseline (speedup 1.0000x reference)
import functools

import jax
import jax.numpy as jnp
from jax import lax
from jax.experimental import pallas as pl
from jax.experimental.pallas import tpu as pltpu
from jax.experimental.pallas import tpu_sc as plsc

ROWS = 128
COLS = 32768
LANES = 16
CHUNKS = COLS // LANES
UNROLL = 2

_info = plsc.get_sparse_core_info()
_NC, _NS = _info.num_cores, _info.num_subcores
NW = _NC * _NS
ROWS_PER_W = ROWS // NW

_mesh = plsc.VectorSubcoreMesh(core_axis_name="c", subcore_axis_name="s")


TILE = 8192
NT = COLS // TILE
CPT = TILE // LANES


@functools.partial(
    pl.kernel,
    mesh=_mesh,
    out_type=jax.ShapeDtypeStruct((ROWS, COLS), jnp.float32),
    scratch_types=(
        [pltpu.VMEM((ROWS_PER_W, TILE), jnp.float32)] * 2
        + [pltpu.SemaphoreType.DMA] * 4
    ),
    compiler_params=pltpu.CompilerParams(needs_layout_passes=False),
)
def _cumsum_sc(x_hbm, out_hbm, buf_a, buf_b, si0, si1, so0, so1):
    bufs = (buf_a, buf_b)
    isems = (si0, si1)
    osems = (so0, so1)
    wid = lax.axis_index("s") * _NC + lax.axis_index("c")
    rows = [wid * ROWS_PER_W + k for k in range(ROWS_PER_W)]

    def start_in(t):
        p = t % 2
        return [
            pltpu.async_copy(
                x_hbm.at[rows[r], pl.ds(t * TILE, TILE)], bufs[p].at[r], isems[p]
            )
            for r in range(ROWS_PER_W)
        ]

    def scan_tile(p, carries):
        @plsc.parallel_loop(0, CPT, carry=carries, unroll=UNROLL)
        def final(i, c):
            off = i * LANES
            c = list(c)
            for r in range(ROWS_PER_W):
                v = bufs[p][r, pl.ds(off, LANES)]
                s = plsc.cumsum(v)
                bufs[p][r, pl.ds(off, LANES)] = s + c[r]
                c[r] = c[r] + jnp.sum(v)
            return tuple(c)

        return final

    carries = tuple(jnp.zeros((LANES,), jnp.float32) for _ in range(ROWS_PER_W))
    in_h, out_h = {}, {}
    in_h[0] = start_in(0)
    for t in range(NT):
        p = t % 2
        if t + 1 < NT:
            if t - 1 >= 0:
                for h in out_h[t - 1]:
                    h.wait()
            in_h[t + 1] = start_in(t + 1)
        for h in in_h[t]:
            h.wait()
        carries = scan_tile(p, carries)
        out_h[t] = [
            pltpu.async_copy(
                bufs[p].at[r], out_hbm.at[rows[r], pl.ds(t * TILE, TILE)], osems[p]
            )
            for r in range(ROWS_PER_W)
        ]
    for t in range(max(0, NT - 2), NT):
        for h in out_h[t]:
            h.wait()


def kernel(x):
    return _cumsum_sc(x)


# speedup vs baseline: 1.2386x; 1.2386x over previous
import functools

import jax
import jax.numpy as jnp
from jax import lax
from jax.experimental import pallas as pl
from jax.experimental.pallas import tpu as pltpu
from jax.experimental.pallas import tpu_sc as plsc

ROWS = 128
COLS = 32768
LANES = 16

_info = plsc.get_sparse_core_info()
_NC, _NS = _info.num_cores, _info.num_subcores
NW = _NC * _NS
ROWS_PER_W = ROWS // NW

TILE = 8192
NT = COLS // TILE
CPT = TILE // LANES
STEP = 8
PIECE = TILE // (CPT // STEP)

_mesh = plsc.VectorSubcoreMesh(core_axis_name="c", subcore_axis_name="s")


@functools.partial(
    pl.kernel,
    mesh=_mesh,
    out_type=jax.ShapeDtypeStruct((ROWS, COLS), jnp.float32),
    scratch_types=(
        [pltpu.VMEM((ROWS_PER_W, TILE), jnp.float32)] * 3
        + [pltpu.SemaphoreType.DMA] * 2
    ),
    compiler_params=pltpu.CompilerParams(needs_layout_passes=False),
)
def _cumsum_sc(x_hbm, out_hbm, buf0, buf1, buf2, isem, osem):
    bufs = (buf0, buf1, buf2)
    wid = lax.axis_index("s") * _NC + lax.axis_index("c")
    rows = [wid * ROWS_PER_W + k for k in range(ROWS_PER_W)]

    def drain(sem):
        pltpu.make_async_copy(
            x_hbm.at[pl.ds(0, ROWS_PER_W), pl.ds(0, TILE)], bufs[0], sem
        ).wait()

    def scan_tile(t, carries):
        p = t % 3
        pin = (t + 1) % 3
        pout = (t - 1) % 3
        buf = bufs[p]

        @plsc.parallel_loop(0, CPT, step=STEP, carry=carries, unroll=1)
        def final(i, c):
            if t + 1 < NT:
                for r in range(ROWS_PER_W):
                    pltpu.async_copy(
                        x_hbm.at[rows[r], pl.ds((t + 1) * TILE + i * LANES, PIECE)],
                        bufs[pin].at[r, pl.ds(i * LANES, PIECE)],
                        isem,
                    )
            if t - 1 >= 0:
                for r in range(ROWS_PER_W):
                    pltpu.async_copy(
                        bufs[pout].at[r, pl.ds(i * LANES, PIECE)],
                        out_hbm.at[rows[r], pl.ds((t - 1) * TILE + i * LANES, PIECE)],
                        osem,
                    )
            c = list(c)
            for u in range(STEP):
                off = (i + u) * LANES
                for r in range(ROWS_PER_W):
                    v = buf[r, pl.ds(off, LANES)]
                    s = plsc.cumsum(v)
                    buf[r, pl.ds(off, LANES)] = s + c[r]
                    c[r] = c[r] + jnp.sum(v)
            return tuple(c)

        return final

    carries = tuple(jnp.zeros((LANES,), jnp.float32) for _ in range(ROWS_PER_W))

    in0 = [
        pltpu.async_copy(x_hbm.at[rows[r], pl.ds(0, TILE)], bufs[0].at[r], isem)
        for r in range(ROWS_PER_W)
    ]
    for h in in0:
        h.wait()

    for t in range(NT):
        if t >= 1:
            drain(isem)
        if t >= 2:
            drain(osem)
        carries = scan_tile(t, carries)

    drain(osem)
    outl = [
        pltpu.async_copy(
            bufs[(NT - 1) % 3].at[r],
            out_hbm.at[rows[r], pl.ds((NT - 1) * TILE, TILE)],
            osem,
        )
        for r in range(ROWS_PER_W)
    ]
    for h in outl:
        h.wait()


def kernel(x):
    return _cumsum_sc(x)
